# Initial kernel scaffold; baseline (speedup 1.0000x reference)
import jax
import jax.numpy as jnp
from jax.experimental import pallas as pl


def kernel(x, edge_index, W1, b1, gamma, beta, W2, b2):
    raise NotImplementedError("write your pallas kernel here")


# speedup vs baseline: 20.9817x; 20.9817x over previous
import functools

import jax
import jax.numpy as jnp
from jax import lax
from jax.experimental import pallas as pl
from jax.experimental.pallas import tpu as pltpu
from jax.experimental.pallas import tpu_sc as plsc

N = 10000
E = 320000
D_IN = 128
D_HID = 32
D_OUT = 128

NPAD = 10240
ROWS_PER_TILE = NPAD // 16

NC = 2
NS = 16
CHUNK = 128
NCHUNKS = E // CHUNK
CHUNKS_PER_CORE = NCHUNKS // NC
ITERS = (CHUNKS_PER_CORE + NS - 1) // NS

_sc_mesh = plsc.VectorSubcoreMesh(core_axis_name="c", subcore_axis_name="s",
                                  num_cores=NC, num_subcores=NS)


def _deg_body(dst_hbm, zeros1_hbm, degp_hbm, idx_v, ones_v, acc):
    c = lax.axis_index("c")
    s = lax.axis_index("s")
    for k in range(CHUNK // 16):
        ones_v[pl.ds(k * 16, 16)] = jnp.ones((16,), jnp.float32)
    pltpu.sync_copy(zeros1_hbm.at[pl.ds(s * ROWS_PER_TILE, ROWS_PER_TILE)],
                    acc.at[pl.ds(s * ROWS_PER_TILE, ROWS_PER_TILE)])
    plsc.subcore_barrier()

    def step(j, carry):
        @pl.when(s + j * NS < CHUNKS_PER_CORE)
        def _():
            base = (c * CHUNKS_PER_CORE + s + j * NS) * CHUNK
            pltpu.sync_copy(dst_hbm.at[pl.ds(base, CHUNK)], idx_v)
            pltpu.sync_copy(ones_v, acc.at[idx_v], add=True)
        return carry

    lax.fori_loop(0, ITERS, step, 0)
    plsc.subcore_barrier()
    pltpu.sync_copy(acc.at[pl.ds(s * ROWS_PER_TILE, ROWS_PER_TILE)],
                    degp_hbm.at[c, pl.ds(s * ROWS_PER_TILE, ROWS_PER_TILE)])


_deg_kernel = functools.partial(
    pl.kernel,
    out_type=jax.ShapeDtypeStruct((NC, NPAD), jnp.float32),
    mesh=_sc_mesh,
    scratch_types=[
        pltpu.VMEM((CHUNK,), jnp.int32),
        pltpu.VMEM((CHUNK,), jnp.float32),
        pltpu.VMEM_SHARED((NPAD,), jnp.float32),
    ],
)(_deg_body)


def _scatter_body(vals_hbm, src_hbm, dst_hbm, zeros2_hbm, out_hbm,
                  si_v, di_v, rows_v, sem, acc):
    c = lax.axis_index("c")
    s = lax.axis_index("s")
    pltpu.sync_copy(zeros2_hbm.at[pl.ds(s * ROWS_PER_TILE, ROWS_PER_TILE)],
                    acc.at[pl.ds(s * ROWS_PER_TILE, ROWS_PER_TILE)])
    plsc.subcore_barrier()

    def step(j, carry):
        @pl.when(s + j * NS < CHUNKS_PER_CORE)
        def _():
            base = (c * CHUNKS_PER_CORE + s + j * NS) * CHUNK
            pltpu.sync_copy(src_hbm.at[pl.ds(base, CHUNK)], si_v)
            pltpu.sync_copy(dst_hbm.at[pl.ds(base, CHUNK)], di_v)
            pltpu.async_copy(vals_hbm.at[si_v], rows_v, sem).wait()
            pltpu.sync_copy(rows_v, acc.at[di_v], add=True)
        return carry

    lax.fori_loop(0, ITERS, step, 0)
    plsc.subcore_barrier()
    pltpu.sync_copy(acc.at[pl.ds(s * ROWS_PER_TILE, ROWS_PER_TILE)],
                    out_hbm.at[c, pl.ds(s * ROWS_PER_TILE, ROWS_PER_TILE)])


_scatter_kernel = functools.partial(
    pl.kernel,
    out_type=jax.ShapeDtypeStruct((NC, NPAD, D_HID), jnp.float32),
    mesh=_sc_mesh,
    compiler_params=pltpu.CompilerParams(use_tc_tiling_on_sc=False),
    scratch_types=[
        pltpu.VMEM((CHUNK,), jnp.int32),
        pltpu.VMEM((CHUNK,), jnp.int32),
        pltpu.VMEM((CHUNK, D_HID), jnp.float32),
        pltpu.SemaphoreType.DMA,
        pltpu.VMEM_SHARED((NPAD, D_HID), jnp.float32),
    ],
)(_scatter_body)


_RB = 1024
_GRID = NPAD // _RB


def _tc1_body(x_ref, w1_ref, degt_ref, y1_ref, dinv_ref):
    deg = degt_ref[:, 0:1] + degt_ref[:, 1:2] + 1.0
    dinv = lax.rsqrt(deg)
    xw = jnp.dot(x_ref[...], w1_ref[...], preferred_element_type=jnp.float32)
    y1_ref[...] = xw * dinv
    dinv_ref[...] = dinv


def _tc1(x_pad, W1, degp_t):
    return pl.pallas_call(
        _tc1_body,
        grid=(_GRID,),
        in_specs=[
            pl.BlockSpec((_RB, D_IN), lambda i: (i, 0)),
            pl.BlockSpec((D_IN, D_HID), lambda i: (0, 0)),
            pl.BlockSpec((_RB, NC), lambda i: (i, 0)),
        ],
        out_specs=[
            pl.BlockSpec((_RB, D_HID), lambda i: (i, 0)),
            pl.BlockSpec((_RB, 1), lambda i: (i, 0)),
        ],
        out_shape=[
            jax.ShapeDtypeStruct((NPAD, D_HID), jnp.float32),
            jax.ShapeDtypeStruct((NPAD, 1), jnp.float32),
        ],
    )(x_pad, W1, degp_t)


def _tc2_body(y1_ref, s1_ref, dinv_ref, b1_ref, g_ref, bt_ref, y2_ref):
    ssum = s1_ref[0] + s1_ref[1] + y1_ref[...]
    h_pre = dinv_ref[...] * ssum + b1_ref[...]
    hv = h_pre[:N]
    mean = jnp.mean(hv, axis=0, keepdims=True)
    var = jnp.mean((hv - mean) ** 2, axis=0, keepdims=True)
    hn = (h_pre - mean) * lax.rsqrt(var + 1e-5) * g_ref[...] + bt_ref[...]
    h = jnp.maximum(hn, 0.0)
    y2_ref[...] = h * dinv_ref[...]


def _tc2(y1, S1, dinv, b1, gamma, beta):
    return pl.pallas_call(
        _tc2_body,
        out_shape=jax.ShapeDtypeStruct((NPAD, D_HID), jnp.float32),
    )(y1, S1, dinv, b1.reshape(1, D_HID), gamma.reshape(1, D_HID),
      beta.reshape(1, D_HID))


def _tc3_body(y2_ref, s2_ref, dinv_ref, w2_ref, b2_ref, out_ref):
    agg = dinv_ref[...] * (s2_ref[0] + s2_ref[1] + y2_ref[...])
    out_ref[...] = jnp.dot(agg, w2_ref[...],
                           preferred_element_type=jnp.float32) + b2_ref[...]


def _tc3(y2, S2, dinv, W2, b2):
    return pl.pallas_call(
        _tc3_body,
        grid=(_GRID,),
        in_specs=[
            pl.BlockSpec((_RB, D_HID), lambda i: (i, 0)),
            pl.BlockSpec((NC, _RB, D_HID), lambda i: (0, i, 0)),
            pl.BlockSpec((_RB, 1), lambda i: (i, 0)),
            pl.BlockSpec((D_HID, D_OUT), lambda i: (0, 0)),
            pl.BlockSpec((1, D_OUT), lambda i: (0, 0)),
        ],
        out_specs=pl.BlockSpec((_RB, D_OUT), lambda i: (i, 0)),
        out_shape=jax.ShapeDtypeStruct((NPAD, D_OUT), jnp.float32),
    )(y2, S2, dinv, W2, b2.reshape(1, D_OUT))


def kernel(x, edge_index, W1, b1, gamma, beta, W2, b2):
    src = edge_index[0]
    dst = edge_index[1]
    x_pad = jnp.pad(x, ((0, NPAD - N), (0, 0)))
    zeros1 = jnp.zeros((NPAD,), jnp.float32)
    zeros2 = jnp.zeros((NPAD, D_HID), jnp.float32)

    degp = _deg_kernel(dst, zeros1)
    y1, dinv = _tc1(x_pad, W1, degp.T)
    S1 = _scatter_kernel(y1, src, dst, zeros2)
    y2 = _tc2(y1, S1, dinv, b1, gamma, beta)
    S2 = _scatter_kernel(y2, src, dst, zeros2)
    out = _tc3(y2, S2, dinv, W2, b2)
    return out[:N]


# speedup vs baseline: 52.6236x; 2.5081x over previous
import functools

import jax
import jax.numpy as jnp
from jax import lax
from jax.experimental import pallas as pl
from jax.experimental.pallas import tpu as pltpu
from jax.experimental.pallas import tpu_sc as plsc

N = 10000
E = 320000
D_IN = 128
D_HID = 32
D_OUT = 128

NPAD = 10240
ROWS_PER_TILE = NPAD // 16

NC = 2
NS = 16
NW = NC * NS
EPT = E // NW
CHUNK = 125
CPT = EPT // CHUNK
GK = 8
NG = CPT // GK

_sc_mesh = plsc.VectorSubcoreMesh(core_axis_name="c", subcore_axis_name="s",
                                  num_cores=NC, num_subcores=NS)


def _deg_body(dstr_hbm, zeros1_hbm, degp_hbm, di_v, ones_v, sem, acc):
    c = lax.axis_index("c")
    s = lax.axis_index("s")
    w = c * NS + s
    for k in range(8):
        ones_v[pl.ds(k * 16, 16)] = jnp.ones((16,), jnp.float32)
    pltpu.sync_copy(zeros1_hbm.at[pl.ds(s * ROWS_PER_TILE, ROWS_PER_TILE)],
                    acc.at[pl.ds(s * ROWS_PER_TILE, ROWS_PER_TILE)])
    pltpu.sync_copy(dstr_hbm.at[w], di_v)
    plsc.subcore_barrier()

    ones = ones_v.at[pl.ds(0, CHUNK)]

    def step(g, carry):
        ds = [pltpu.async_copy(ones, acc.at[di_v.at[g * GK + b]], sem,
                               add=True) for b in range(GK)]
        for d in ds:
            d.wait()
        return carry

    lax.fori_loop(0, NG, step, 0)
    plsc.subcore_barrier()
    pltpu.sync_copy(acc.at[pl.ds(s * ROWS_PER_TILE, ROWS_PER_TILE)],
                    degp_hbm.at[c, pl.ds(s * ROWS_PER_TILE, ROWS_PER_TILE)])


_deg_kernel = functools.partial(
    pl.kernel,
    out_type=jax.ShapeDtypeStruct((NC, NPAD), jnp.float32),
    mesh=_sc_mesh,
    compiler_params=pltpu.CompilerParams(use_tc_tiling_on_sc=False),
    scratch_types=[
        pltpu.VMEM((CPT, CHUNK), jnp.int32),
        pltpu.VMEM((128,), jnp.float32),
        pltpu.SemaphoreType.DMA,
        pltpu.VMEM_SHARED((NPAD,), jnp.float32),
    ],
)(_deg_body)


def _scatter_body(vals_hbm, srcr_hbm, dstr_hbm, zeros2_hbm, out_hbm,
                  si_v, di_v, rows_v, gsem, ssem, acc):
    c = lax.axis_index("c")
    s = lax.axis_index("s")
    w = c * NS + s
    pltpu.sync_copy(zeros2_hbm.at[pl.ds(s * ROWS_PER_TILE, ROWS_PER_TILE)],
                    acc.at[pl.ds(s * ROWS_PER_TILE, ROWS_PER_TILE)])
    pltpu.sync_copy(srcr_hbm.at[w], si_v)
    pltpu.sync_copy(dstr_hbm.at[w], di_v)
    plsc.subcore_barrier()

    def step(g, carry):
        gs = [pltpu.async_copy(vals_hbm.at[si_v.at[g * GK + b]],
                               rows_v.at[b], gsem) for b in range(GK)]
        ss = []
        for b in range(GK):
            gs[b].wait()
            ss.append(pltpu.async_copy(rows_v.at[b],
                                       acc.at[di_v.at[g * GK + b]],
                                       ssem, add=True))
        for d in ss:
            d.wait()
        return carry

    lax.fori_loop(0, NG, step, 0)
    plsc.subcore_barrier()
    pltpu.sync_copy(acc.at[pl.ds(s * ROWS_PER_TILE, ROWS_PER_TILE)],
                    out_hbm.at[c, pl.ds(s * ROWS_PER_TILE, ROWS_PER_TILE)])


_scatter_kernel = functools.partial(
    pl.kernel,
    out_type=jax.ShapeDtypeStruct((NC, NPAD, D_HID), jnp.float32),
    mesh=_sc_mesh,
    compiler_params=pltpu.CompilerParams(use_tc_tiling_on_sc=False),
    scratch_types=[
        pltpu.VMEM((CPT, CHUNK), jnp.int32),
        pltpu.VMEM((CPT, CHUNK), jnp.int32),
        pltpu.VMEM((GK, CHUNK, D_HID), jnp.float32),
        pltpu.SemaphoreType.DMA,
        pltpu.SemaphoreType.DMA,
        pltpu.VMEM_SHARED((NPAD, D_HID), jnp.float32),
    ],
)(_scatter_body)


_RB = 1024
_GRID = NPAD // _RB


def _tc1_body(x_ref, w1_ref, degt_ref, y1_ref, dinv_ref):
    deg = degt_ref[:, 0:1] + degt_ref[:, 1:2] + 1.0
    dinv = lax.rsqrt(deg)
    xw = jnp.dot(x_ref[...], w1_ref[...], preferred_element_type=jnp.float32)
    y1_ref[...] = xw * dinv
    dinv_ref[...] = dinv


def _tc1(x_pad, W1, degp_t):
    return pl.pallas_call(
        _tc1_body,
        grid=(_GRID,),
        in_specs=[
            pl.BlockSpec((_RB, D_IN), lambda i: (i, 0)),
            pl.BlockSpec((D_IN, D_HID), lambda i: (0, 0)),
            pl.BlockSpec((_RB, NC), lambda i: (i, 0)),
        ],
        out_specs=[
            pl.BlockSpec((_RB, D_HID), lambda i: (i, 0)),
            pl.BlockSpec((_RB, 1), lambda i: (i, 0)),
        ],
        out_shape=[
            jax.ShapeDtypeStruct((NPAD, D_HID), jnp.float32),
            jax.ShapeDtypeStruct((NPAD, 1), jnp.float32),
        ],
    )(x_pad, W1, degp_t)


def _tc2_body(y1_ref, s1_ref, dinv_ref, b1_ref, g_ref, bt_ref, y2_ref):
    ssum = s1_ref[0] + s1_ref[1] + y1_ref[...]
    h_pre = dinv_ref[...] * ssum + b1_ref[...]
    hv = h_pre[:N]
    mean = jnp.mean(hv, axis=0, keepdims=True)
    var = jnp.mean((hv - mean) ** 2, axis=0, keepdims=True)
    hn = (h_pre - mean) * lax.rsqrt(var + 1e-5) * g_ref[...] + bt_ref[...]
    h = jnp.maximum(hn, 0.0)
    y2_ref[...] = h * dinv_ref[...]


def _tc2(y1, S1, dinv, b1, gamma, beta):
    return pl.pallas_call(
        _tc2_body,
        out_shape=jax.ShapeDtypeStruct((NPAD, D_HID), jnp.float32),
    )(y1, S1, dinv, b1.reshape(1, D_HID), gamma.reshape(1, D_HID),
      beta.reshape(1, D_HID))


def _tc3_body(y2_ref, s2_ref, dinv_ref, w2_ref, b2_ref, out_ref):
    agg = dinv_ref[...] * (s2_ref[0] + s2_ref[1] + y2_ref[...])
    out_ref[...] = jnp.dot(agg, w2_ref[...],
                           preferred_element_type=jnp.float32) + b2_ref[...]


def _tc3(y2, S2, dinv, W2, b2):
    return pl.pallas_call(
        _tc3_body,
        grid=(_GRID,),
        in_specs=[
            pl.BlockSpec((_RB, D_HID), lambda i: (i, 0)),
            pl.BlockSpec((NC, _RB, D_HID), lambda i: (0, i, 0)),
            pl.BlockSpec((_RB, 1), lambda i: (i, 0)),
            pl.BlockSpec((D_HID, D_OUT), lambda i: (0, 0)),
            pl.BlockSpec((1, D_OUT), lambda i: (0, 0)),
        ],
        out_specs=pl.BlockSpec((_RB, D_OUT), lambda i: (i, 0)),
        out_shape=jax.ShapeDtypeStruct((NPAD, D_OUT), jnp.float32),
    )(y2, S2, dinv, W2, b2.reshape(1, D_OUT))


def kernel(x, edge_index, W1, b1, gamma, beta, W2, b2):
    src_r = edge_index[0].reshape(NW, CPT, CHUNK)
    dst_r = edge_index[1].reshape(NW, CPT, CHUNK)
    x_pad = jnp.pad(x, ((0, NPAD - N), (0, 0)))
    zeros1 = jnp.zeros((NPAD,), jnp.float32)
    zeros2 = jnp.zeros((NPAD, D_HID), jnp.float32)

    degp = _deg_kernel(dst_r, zeros1)
    y1, dinv = _tc1(x_pad, W1, degp.T)
    S1 = _scatter_kernel(y1, src_r, dst_r, zeros2)
    y2 = _tc2(y1, S1, dinv, b1, gamma, beta)
    S2 = _scatter_kernel(y2, src_r, dst_r, zeros2)
    out = _tc3(y2, S2, dinv, W2, b2)
    return out[:N]


# speedup vs baseline: 62.8013x; 1.1934x over previous
import functools

import jax
import jax.numpy as jnp
from jax import lax
from jax.experimental import pallas as pl
from jax.experimental.pallas import tpu as pltpu
from jax.experimental.pallas import tpu_sc as plsc

N = 10000
E = 320000
D_IN = 128
D_HID = 32
D_OUT = 128

NPAD = 10240
ROWS_PER_TILE = NPAD // 16
NP4 = NPAD // 4
NV4 = N // 4

NC = 2
NS = 16
NW = NC * NS
EPT = E // NW
CHUNK = 125
CPT = EPT // CHUNK
GK = 8
NG = CPT // GK

_sc_mesh = plsc.VectorSubcoreMesh(core_axis_name="c", subcore_axis_name="s",
                                  num_cores=NC, num_subcores=NS)


def _deg_body(er_hbm, zeros1_hbm, degp_hbm, di_v, ones_v, sem, acc):
    c = lax.axis_index("c")
    s = lax.axis_index("s")
    w = c * NS + s
    for k in range(8):
        ones_v[pl.ds(k * 16, 16)] = jnp.ones((16,), jnp.float32)
    pltpu.sync_copy(zeros1_hbm.at[pl.ds(s * ROWS_PER_TILE, ROWS_PER_TILE)],
                    acc.at[pl.ds(s * ROWS_PER_TILE, ROWS_PER_TILE)])
    pltpu.sync_copy(er_hbm.at[0, w], di_v)
    plsc.subcore_barrier()

    ones = ones_v.at[pl.ds(0, CHUNK)]

    def step(g, carry):
        ds = [pltpu.async_copy(ones, acc.at[di_v.at[g * GK + b]], sem,
                               add=True) for b in range(GK)]
        for d in ds:
            d.wait()
        return carry

    lax.fori_loop(0, NG, step, 0)
    plsc.subcore_barrier()
    pltpu.sync_copy(acc.at[pl.ds(s * ROWS_PER_TILE, ROWS_PER_TILE)],
                    degp_hbm.at[c, pl.ds(s * ROWS_PER_TILE, ROWS_PER_TILE)])


_deg_kernel = functools.partial(
    pl.kernel,
    out_type=jax.ShapeDtypeStruct((NC, NPAD), jnp.float32),
    mesh=_sc_mesh,
    compiler_params=pltpu.CompilerParams(use_tc_tiling_on_sc=False),
    scratch_types=[
        pltpu.VMEM((CPT, CHUNK), jnp.int32),
        pltpu.VMEM((128,), jnp.float32),
        pltpu.SemaphoreType.DMA,
        pltpu.VMEM_SHARED((NPAD,), jnp.float32),
    ],
)(_deg_body)


def _scatter_body(vals_hbm, er_hbm, zeros2_hbm, out_hbm,
                  si_v, di_v, rows_v, gsem, ssem, acc):
    c = lax.axis_index("c")
    s = lax.axis_index("s")
    w = c * NS + s
    pltpu.sync_copy(zeros2_hbm.at[pl.ds(s * ROWS_PER_TILE, ROWS_PER_TILE)],
                    acc.at[pl.ds(s * ROWS_PER_TILE, ROWS_PER_TILE)])
    pltpu.sync_copy(er_hbm.at[0, w], si_v)
    pltpu.sync_copy(er_hbm.at[1, w], di_v)
    plsc.subcore_barrier()

    def step(g, carry):
        gs = [pltpu.async_copy(vals_hbm.at[si_v.at[g * GK + b]],
                               rows_v.at[b], gsem) for b in range(GK)]
        ss = []
        for b in range(GK):
            gs[b].wait()
            ss.append(pltpu.async_copy(rows_v.at[b],
                                       acc.at[di_v.at[g * GK + b]],
                                       ssem, add=True))
        for d in ss:
            d.wait()
        return carry

    lax.fori_loop(0, NG, step, 0)
    plsc.subcore_barrier()
    pltpu.sync_copy(acc.at[pl.ds(s * ROWS_PER_TILE, ROWS_PER_TILE)],
                    out_hbm.at[c, pl.ds(s * ROWS_PER_TILE, ROWS_PER_TILE)])


_scatter_kernel = functools.partial(
    pl.kernel,
    out_type=jax.ShapeDtypeStruct((NC, NPAD, D_HID), jnp.float32),
    mesh=_sc_mesh,
    compiler_params=pltpu.CompilerParams(use_tc_tiling_on_sc=False),
    scratch_types=[
        pltpu.VMEM((CPT, CHUNK), jnp.int32),
        pltpu.VMEM((CPT, CHUNK), jnp.int32),
        pltpu.VMEM((GK, CHUNK, D_HID), jnp.float32),
        pltpu.SemaphoreType.DMA,
        pltpu.SemaphoreType.DMA,
        pltpu.VMEM_SHARED((NPAD, D_HID), jnp.float32),
    ],
)(_scatter_body)


_PB = 256
_GRID = NP4 // _PB


def _dinv_col(degt):
    return lax.rsqrt(degt[:, 0:1] + degt[:, 1:2] + 1.0)


def _tc1_body(x0, x1, x2, x3, w1_ref, d0, d1, d2, d3, y1p_ref):
    xs = (x0, x1, x2, x3)
    ds = (d0, d1, d2, d3)
    cols = []
    for a in range(4):
        xw = jnp.dot(xs[a][...], w1_ref[...],
                     preferred_element_type=jnp.float32)
        cols.append(xw * jnp.broadcast_to(_dinv_col(ds[a][...]),
                                          (_PB, D_HID)))
    y1p_ref[...] = jnp.concatenate(cols, axis=1)


def _tc1(x, W1, degt):
    xspec = lambda a: pl.BlockSpec((_PB, D_IN), lambda i, a=a: (10 * a + i, 0))
    dspec = lambda a: pl.BlockSpec((_PB, NC), lambda i, a=a: (10 * a + i, 0))
    return pl.pallas_call(
        _tc1_body,
        grid=(_GRID,),
        in_specs=[xspec(0), xspec(1), xspec(2), xspec(3),
                  pl.BlockSpec((D_IN, D_HID), lambda i: (0, 0)),
                  dspec(0), dspec(1), dspec(2), dspec(3)],
        out_specs=pl.BlockSpec((_PB, 128), lambda i: (i, 0)),
        out_shape=jax.ShapeDtypeStruct((NP4, 128), jnp.float32),
    )(x, x, x, x, W1, degt, degt, degt, degt)


def _tc2_body(y1p_ref, s1p_ref, degt_ref, b1_ref, g_ref, bt_ref, y2p_ref):
    dg = degt_ref[...]
    dinvp = jnp.concatenate(
        [jnp.broadcast_to(_dinv_col(dg[2560 * a:2560 * (a + 1)]),
                          (NP4, D_HID)) for a in range(4)], axis=1)
    sp = s1p_ref[0] + s1p_ref[1] + y1p_ref[...]
    h_pre = dinvp * sp + b1_ref[...]
    rows = lax.broadcasted_iota(jnp.int32, (NP4, 128), 0)
    lanes = lax.broadcasted_iota(jnp.int32, (NP4, 128), 1)
    valid = jnp.logical_or(lanes < 96, rows < (N - 3 * 2560))
    hv = jnp.where(valid, h_pre, 0.0)
    s4 = jnp.sum(hv, axis=0, keepdims=True)
    mean = (s4[:, 0:32] + s4[:, 32:64] + s4[:, 64:96] + s4[:, 96:128]) / N
    meanp = jnp.concatenate([mean] * 4, axis=1)
    d = jnp.where(valid, h_pre - meanp, 0.0)
    v4 = jnp.sum(d * d, axis=0, keepdims=True)
    var = (v4[:, 0:32] + v4[:, 32:64] + v4[:, 64:96] + v4[:, 96:128]) / N
    varp = jnp.concatenate([var] * 4, axis=1)
    hn = (h_pre - meanp) * lax.rsqrt(varp + 1e-5) * g_ref[...] + bt_ref[...]
    y2p_ref[...] = jnp.maximum(hn, 0.0) * dinvp


def _tc2(y1p, S1p, degt, b1, gamma, beta):
    t = lambda v: jnp.tile(v, 4).reshape(1, 128)
    return pl.pallas_call(
        _tc2_body,
        out_shape=jax.ShapeDtypeStruct((NP4, 128), jnp.float32),
    )(y1p, S1p, degt, t(b1), t(gamma), t(beta))


def _tc3_body(y2p_ref, s2p_ref, d0, d1, d2, d3, w2_ref, b2_ref, out_ref):
    ds = (d0, d1, d2, d3)
    for a in range(4):
        sl = slice(32 * a, 32 * (a + 1))
        agg = (y2p_ref[:, sl] + s2p_ref[0][:, sl] + s2p_ref[1][:, sl])
        agg = agg * jnp.broadcast_to(_dinv_col(ds[a][...]), (_PB, D_HID))
        out_ref[a] = jnp.dot(agg, w2_ref[...],
                             preferred_element_type=jnp.float32) + b2_ref[...]


def _tc3(y2p, S2p, degt, W2, b2):
    dspec = lambda a: pl.BlockSpec((_PB, NC), lambda i, a=a: (10 * a + i, 0))
    return pl.pallas_call(
        _tc3_body,
        grid=(_GRID,),
        in_specs=[
            pl.BlockSpec((_PB, 128), lambda i: (i, 0)),
            pl.BlockSpec((NC, _PB, 128), lambda i: (0, i, 0)),
            dspec(0), dspec(1), dspec(2), dspec(3),
            pl.BlockSpec((D_HID, D_OUT), lambda i: (0, 0)),
            pl.BlockSpec((1, D_OUT), lambda i: (0, 0)),
        ],
        out_specs=pl.BlockSpec((4, _PB, D_OUT), lambda i: (0, i, 0)),
        out_shape=jax.ShapeDtypeStruct((4, NP4, D_OUT), jnp.float32),
    )(y2p, S2p, degt, degt, degt, degt, W2, b2.reshape(1, D_OUT))


def kernel(x, edge_index, W1, b1, gamma, beta, W2, b2):
    src, dst = edge_index[0], edge_index[1]

    def perm(n):
        a = ((n >> 9) * 6554) >> 15
        return 4 * (n - a * NP4) + a

    srcp = perm(src)
    dstp = perm(dst)
    er = jnp.stack([srcp, dstp]).reshape(2, NW, CPT, CHUNK)
    erd = dst.reshape(1, NW, CPT, CHUNK)
    zeros1 = jnp.zeros((NPAD,), jnp.float32)
    zeros2 = jnp.zeros((NPAD, D_HID), jnp.float32)

    degp = _deg_kernel(erd, zeros1)
    degt = degp.T
    y1p = _tc1(x, W1, degt)
    y1 = y1p.reshape(NPAD, D_HID)
    S1 = _scatter_kernel(y1, er, zeros2)
    S1p = S1.reshape(NC, NP4, 128)
    y2p = _tc2(y1p, S1p, degt, b1, gamma, beta)
    y2 = y2p.reshape(NPAD, D_HID)
    S2 = _scatter_kernel(y2, er, zeros2)
    S2p = S2.reshape(NC, NP4, 128)
    out4 = _tc3(y2p, S2p, degt, W2, b2)
    return out4.reshape(NPAD, D_OUT)[:N]


# speedup vs baseline: 71.5650x; 1.1395x over previous
import functools

import jax
import jax.numpy as jnp
from jax import lax
from jax.experimental import pallas as pl
from jax.experimental.pallas import tpu as pltpu
from jax.experimental.pallas import tpu_sc as plsc

N = 10000
E = 320000
D_IN = 128
D_HID = 32
D_OUT = 128

NPAD = 10240
ROWS_PER_TILE = NPAD // 16
NP4 = NPAD // 4
NV4 = N // 4

NC = 2
NS = 16
NW = NC * NS
EPT = E // NW
CHUNK = 125
CPT = EPT // CHUNK
GK = 8
NG = CPT // GK

_sc_mesh = plsc.VectorSubcoreMesh(core_axis_name="c", subcore_axis_name="s",
                                  num_cores=NC, num_subcores=NS)


def _deg_body(er_hbm, zeros1_hbm, degp_hbm, di_v, ones_v, sem, acc):
    c = lax.axis_index("c")
    s = lax.axis_index("s")
    w = c * NS + s
    for k in range(8):
        ones_v[pl.ds(k * 16, 16)] = jnp.ones((16,), jnp.float32)
    pltpu.sync_copy(zeros1_hbm.at[pl.ds(s * ROWS_PER_TILE, ROWS_PER_TILE)],
                    acc.at[pl.ds(s * ROWS_PER_TILE, ROWS_PER_TILE)])
    pltpu.sync_copy(er_hbm.at[1, w], di_v)
    plsc.subcore_barrier()

    ones = ones_v.at[pl.ds(0, CHUNK)]

    def step(g, carry):
        ds = [pltpu.async_copy(ones, acc.at[di_v.at[g * GK + b]], sem,
                               add=True) for b in range(GK)]
        for d in ds:
            d.wait()
        return carry

    lax.fori_loop(0, NG, step, 0)
    plsc.subcore_barrier()
    pltpu.sync_copy(acc.at[pl.ds(s * ROWS_PER_TILE, ROWS_PER_TILE)],
                    degp_hbm.at[c, pl.ds(s * ROWS_PER_TILE, ROWS_PER_TILE)])


_deg_kernel = functools.partial(
    pl.kernel,
    out_type=jax.ShapeDtypeStruct((NC, NPAD), jnp.float32),
    mesh=_sc_mesh,
    compiler_params=pltpu.CompilerParams(use_tc_tiling_on_sc=False),
    scratch_types=[
        pltpu.VMEM((CPT, CHUNK), jnp.int32),
        pltpu.VMEM((128,), jnp.float32),
        pltpu.SemaphoreType.DMA,
        pltpu.VMEM_SHARED((NPAD,), jnp.float32),
    ],
)(_deg_body)


def _scatter_body(vals_hbm, er_hbm, zeros2_hbm, out_hbm,
                  si_v, di_v, rows_v, gsem, ssem, acc):
    c = lax.axis_index("c")
    s = lax.axis_index("s")
    w = c * NS + s
    pltpu.sync_copy(zeros2_hbm.at[pl.ds(s * ROWS_PER_TILE, ROWS_PER_TILE)],
                    acc.at[pl.ds(s * ROWS_PER_TILE, ROWS_PER_TILE)])
    pltpu.sync_copy(er_hbm.at[0, w], si_v)
    pltpu.sync_copy(er_hbm.at[1, w], di_v)
    plsc.subcore_barrier()

    def step(g, carry):
        gs = [pltpu.async_copy(vals_hbm.at[si_v.at[g * GK + b]],
                               rows_v.at[b], gsem) for b in range(GK)]
        ss = []
        for b in range(GK):
            gs[b].wait()
            ss.append(pltpu.async_copy(rows_v.at[b],
                                       acc.at[di_v.at[g * GK + b]],
                                       ssem, add=True))
        for d in ss:
            d.wait()
        return carry

    lax.fori_loop(0, NG, step, 0)
    plsc.subcore_barrier()
    pltpu.sync_copy(acc.at[pl.ds(s * ROWS_PER_TILE, ROWS_PER_TILE)],
                    out_hbm.at[c, pl.ds(s * ROWS_PER_TILE, ROWS_PER_TILE)])


_scatter_kernel = functools.partial(
    pl.kernel,
    out_type=jax.ShapeDtypeStruct((NC, NPAD, D_HID), jnp.float32),
    mesh=_sc_mesh,
    compiler_params=pltpu.CompilerParams(use_tc_tiling_on_sc=False),
    scratch_types=[
        pltpu.VMEM((CPT, CHUNK), jnp.int32),
        pltpu.VMEM((CPT, CHUNK), jnp.int32),
        pltpu.VMEM((GK, CHUNK, D_HID), jnp.float32),
        pltpu.SemaphoreType.DMA,
        pltpu.SemaphoreType.DMA,
        pltpu.VMEM_SHARED((NPAD, D_HID), jnp.float32),
    ],
)(_scatter_body)


NQ = N // 4


def _dinv4(deg4):
    return lax.rsqrt(deg4 + 1.0)


def _tc1_body(x4_ref, w1_ref, deg4_ref, y1p_ref):
    dinv = _dinv4(deg4_ref[0][:NQ] + deg4_ref[1][:NQ])
    cols = []
    for a in range(4):
        xw = jnp.dot(x4_ref[:, 128 * a:128 * (a + 1)], w1_ref[...],
                     preferred_element_type=jnp.float32)
        cols.append(xw * jnp.broadcast_to(dinv[:, a:a + 1], (NQ, D_HID)))
    y1p_ref[...] = jnp.concatenate(cols, axis=1)


def _tc1(x4, W1, deg4):
    return pl.pallas_call(
        _tc1_body,
        out_shape=jax.ShapeDtypeStruct((NQ, 128), jnp.float32),
    )(x4, W1, deg4)


def _tc2_body(y1p_ref, s1p_ref, deg4_ref, b1_ref, g_ref, bt_ref, y2p_ref):
    dinv = _dinv4(deg4_ref[0][:NQ] + deg4_ref[1][:NQ])
    dinvp = jnp.concatenate(
        [jnp.broadcast_to(dinv[:, a:a + 1], (NQ, D_HID)) for a in range(4)],
        axis=1)
    sp = s1p_ref[0][:NQ] + s1p_ref[1][:NQ] + y1p_ref[...]
    h_pre = dinvp * sp + b1_ref[...]
    s4 = jnp.sum(h_pre, axis=0, keepdims=True)
    mean = (s4[:, 0:32] + s4[:, 32:64] + s4[:, 64:96] + s4[:, 96:128]) / N
    meanp = jnp.concatenate([mean] * 4, axis=1)
    d = h_pre - meanp
    v4 = jnp.sum(d * d, axis=0, keepdims=True)
    var = (v4[:, 0:32] + v4[:, 32:64] + v4[:, 64:96] + v4[:, 96:128]) / N
    varp = jnp.concatenate([var] * 4, axis=1)
    hn = d * lax.rsqrt(varp + 1e-5) * g_ref[...] + bt_ref[...]
    y2p_ref[...] = jnp.maximum(hn, 0.0) * dinvp


def _tc2(y1p, S1p, deg4, b1, gamma, beta):
    t = lambda v: jnp.tile(v, 4).reshape(1, 128)
    return pl.pallas_call(
        _tc2_body,
        out_shape=jax.ShapeDtypeStruct((NQ, 128), jnp.float32),
    )(y1p, S1p, deg4, t(b1), t(gamma), t(beta))


def _tc3_body(y2p_ref, s2p_ref, deg4_ref, w2_ref, b2_ref, out4_ref):
    dinv = _dinv4(deg4_ref[0][:NQ] + deg4_ref[1][:NQ])
    cols = []
    for a in range(4):
        sl = slice(32 * a, 32 * (a + 1))
        agg = y2p_ref[:, sl] + s2p_ref[0][:NQ, sl] + s2p_ref[1][:NQ, sl]
        agg = agg * jnp.broadcast_to(dinv[:, a:a + 1], (NQ, D_HID))
        cols.append(jnp.dot(agg, w2_ref[...],
                            preferred_element_type=jnp.float32)
                    + b2_ref[...])
    out4_ref[...] = jnp.concatenate(cols, axis=1)


def _tc3(y2p, S2p, deg4, W2, b2):
    return pl.pallas_call(
        _tc3_body,
        out_shape=jax.ShapeDtypeStruct((NQ, 4 * D_OUT), jnp.float32),
    )(y2p, S2p, deg4, W2, b2.reshape(1, D_OUT))


def kernel(x, edge_index, W1, b1, gamma, beta, W2, b2):
    er = edge_index.reshape(2, NW, CPT, CHUNK)
    x4 = x.reshape(NQ, 4 * D_IN)
    zeros1 = jnp.zeros((NPAD,), jnp.float32)
    zeros2 = jnp.zeros((NPAD, D_HID), jnp.float32)

    degp = _deg_kernel(er, zeros1)
    deg4 = degp.reshape(NC, NP4, 4)
    y1p = _tc1(x4, W1, deg4)
    y1 = y1p.reshape(N, D_HID)
    S1 = _scatter_kernel(y1, er, zeros2)
    S1p = S1.reshape(NC, NP4, 128)
    y2p = _tc2(y1p, S1p, deg4, b1, gamma, beta)
    y2 = y2p.reshape(N, D_HID)
    S2 = _scatter_kernel(y2, er, zeros2)
    S2p = S2.reshape(NC, NP4, 128)
    out4 = _tc3(y2p, S2p, deg4, W2, b2)
    return out4.reshape(N, D_OUT)


# speedup vs baseline: 77.7454x; 1.0864x over previous
import functools

import jax
import jax.numpy as jnp
from jax import lax
from jax.experimental import pallas as pl
from jax.experimental.pallas import tpu as pltpu
from jax.experimental.pallas import tpu_sc as plsc

N = 10000
E = 320000
D_IN = 128
D_HID = 32
D_OUT = 128

NPAD = 10240
ROWS_PER_TILE = NPAD // 16
NP4 = NPAD // 4
NV4 = N // 4

NC = 2
NS = 16
NW = NC * NS
EPT = E // NW
CHUNK = 125
CPT = EPT // CHUNK
GK = 8
NG = CPT // GK

_sc_mesh = plsc.VectorSubcoreMesh(core_axis_name="c", subcore_axis_name="s",
                                  num_cores=NC, num_subcores=NS)


def _deg_body(er_hbm, zeros1_hbm, degp_hbm, di_v, ones_v, sem, acc):
    c = lax.axis_index("c")
    s = lax.axis_index("s")
    w = c * NS + s
    for k in range(8):
        ones_v[pl.ds(k * 16, 16)] = jnp.ones((16,), jnp.float32)
    pltpu.sync_copy(zeros1_hbm.at[pl.ds(s * ROWS_PER_TILE, ROWS_PER_TILE)],
                    acc.at[pl.ds(s * ROWS_PER_TILE, ROWS_PER_TILE)])
    pltpu.sync_copy(er_hbm.at[1, w], di_v)
    plsc.subcore_barrier()

    ones = ones_v.at[pl.ds(0, CHUNK)]

    def step(g, carry):
        ds = [pltpu.async_copy(ones, acc.at[di_v.at[g * GK + b]], sem,
                               add=True) for b in range(GK)]
        for d in ds:
            d.wait()
        return carry

    lax.fori_loop(0, NG, step, 0)
    plsc.subcore_barrier()
    pltpu.sync_copy(acc.at[pl.ds(s * ROWS_PER_TILE, ROWS_PER_TILE)],
                    degp_hbm.at[c, pl.ds(s * ROWS_PER_TILE, ROWS_PER_TILE)])


_deg_kernel = functools.partial(
    pl.kernel,
    out_type=jax.ShapeDtypeStruct((NC, NPAD), jnp.float32),
    mesh=_sc_mesh,
    compiler_params=pltpu.CompilerParams(use_tc_tiling_on_sc=False),
    scratch_types=[
        pltpu.VMEM((CPT, CHUNK), jnp.int32),
        pltpu.VMEM((128,), jnp.float32),
        pltpu.SemaphoreType.DMA,
        pltpu.VMEM_SHARED((NPAD,), jnp.float32),
    ],
)(_deg_body)


def _scatter_body(vals_hbm, er_hbm, zeros2_hbm, out_hbm,
                  si_v, di_v, rows_v, gsem, ssem, acc):
    c = lax.axis_index("c")
    s = lax.axis_index("s")
    w = c * NS + s
    pltpu.sync_copy(zeros2_hbm.at[pl.ds(s * ROWS_PER_TILE, ROWS_PER_TILE)],
                    acc.at[pl.ds(s * ROWS_PER_TILE, ROWS_PER_TILE)])
    pltpu.sync_copy(er_hbm.at[0, w], si_v)
    pltpu.sync_copy(er_hbm.at[1, w], di_v)
    plsc.subcore_barrier()

    def wait_g(bank, b):
        pltpu.make_async_copy(vals_hbm.at[si_v.at[0]],
                              rows_v.at[bank, b], gsem).wait()

    def wait_s(bank, b):
        pltpu.make_async_copy(rows_v.at[bank, b],
                              acc.at[di_v.at[0]], ssem).wait()

    for b in range(GK):
        pltpu.async_copy(vals_hbm.at[si_v.at[b]], rows_v.at[0, b], gsem)

    def step(g, carry):
        p = lax.rem(g, 2)
        po = 1 - p

        @pl.when(g > 0)
        def _():
            for b in range(GK):
                wait_s(po, b)

        @pl.when(g < NG - 1)
        def _():
            for b in range(GK):
                pltpu.async_copy(vals_hbm.at[si_v.at[(g + 1) * GK + b]],
                                 rows_v.at[po, b], gsem)

        for b in range(GK):
            wait_g(p, b)
            pltpu.async_copy(rows_v.at[p, b], acc.at[di_v.at[g * GK + b]],
                             ssem, add=True)
        return carry

    lax.fori_loop(0, NG, step, 0)
    for b in range(GK):
        wait_s((NG - 1) % 2, b)
    plsc.subcore_barrier()
    pltpu.sync_copy(acc.at[pl.ds(s * ROWS_PER_TILE, ROWS_PER_TILE)],
                    out_hbm.at[c, pl.ds(s * ROWS_PER_TILE, ROWS_PER_TILE)])


_scatter_kernel = functools.partial(
    pl.kernel,
    out_type=jax.ShapeDtypeStruct((NC, NPAD, D_HID), jnp.float32),
    mesh=_sc_mesh,
    compiler_params=pltpu.CompilerParams(use_tc_tiling_on_sc=False),
    scratch_types=[
        pltpu.VMEM((CPT, CHUNK), jnp.int32),
        pltpu.VMEM((CPT, CHUNK), jnp.int32),
        pltpu.VMEM((2, GK, CHUNK, D_HID), jnp.float32),
        pltpu.SemaphoreType.DMA,
        pltpu.SemaphoreType.DMA,
        pltpu.VMEM_SHARED((NPAD, D_HID), jnp.float32),
    ],
)(_scatter_body)


NQ = N // 4


def _dinv4(deg4):
    return lax.rsqrt(deg4 + 1.0)


def _tc1_body(x4_ref, w1_ref, deg4_ref, y1p_ref):
    dinv = _dinv4(deg4_ref[0][:NQ] + deg4_ref[1][:NQ])
    cols = []
    for a in range(4):
        xw = jnp.dot(x4_ref[:, 128 * a:128 * (a + 1)], w1_ref[...],
                     preferred_element_type=jnp.float32)
        cols.append(xw * jnp.broadcast_to(dinv[:, a:a + 1], (NQ, D_HID)))
    y1p_ref[...] = jnp.concatenate(cols, axis=1)


def _tc1(x4, W1, deg4):
    return pl.pallas_call(
        _tc1_body,
        out_shape=jax.ShapeDtypeStruct((NQ, 128), jnp.float32),
    )(x4, W1, deg4)


def _tc2_body(y1p_ref, s1p_ref, deg4_ref, b1_ref, g_ref, bt_ref, y2p_ref):
    dinv = _dinv4(deg4_ref[0][:NQ] + deg4_ref[1][:NQ])
    dinvp = jnp.concatenate(
        [jnp.broadcast_to(dinv[:, a:a + 1], (NQ, D_HID)) for a in range(4)],
        axis=1)
    sp = s1p_ref[0][:NQ] + s1p_ref[1][:NQ] + y1p_ref[...]
    h_pre = dinvp * sp + b1_ref[...]
    s4 = jnp.sum(h_pre, axis=0, keepdims=True)
    mean = (s4[:, 0:32] + s4[:, 32:64] + s4[:, 64:96] + s4[:, 96:128]) / N
    meanp = jnp.concatenate([mean] * 4, axis=1)
    d = h_pre - meanp
    v4 = jnp.sum(d * d, axis=0, keepdims=True)
    var = (v4[:, 0:32] + v4[:, 32:64] + v4[:, 64:96] + v4[:, 96:128]) / N
    varp = jnp.concatenate([var] * 4, axis=1)
    hn = d * lax.rsqrt(varp + 1e-5) * g_ref[...] + bt_ref[...]
    y2p_ref[...] = jnp.maximum(hn, 0.0) * dinvp


def _tc2(y1p, S1p, deg4, b1, gamma, beta):
    t = lambda v: jnp.tile(v, 4).reshape(1, 128)
    return pl.pallas_call(
        _tc2_body,
        out_shape=jax.ShapeDtypeStruct((NQ, 128), jnp.float32),
    )(y1p, S1p, deg4, t(b1), t(gamma), t(beta))


def _tc3_body(y2p_ref, s2p_ref, deg4_ref, w2_ref, b2_ref, out4_ref):
    dinv = _dinv4(deg4_ref[0][:NQ] + deg4_ref[1][:NQ])
    cols = []
    for a in range(4):
        sl = slice(32 * a, 32 * (a + 1))
        agg = y2p_ref[:, sl] + s2p_ref[0][:NQ, sl] + s2p_ref[1][:NQ, sl]
        agg = agg * jnp.broadcast_to(dinv[:, a:a + 1], (NQ, D_HID))
        cols.append(jnp.dot(agg, w2_ref[...],
                            preferred_element_type=jnp.float32)
                    + b2_ref[...])
    out4_ref[...] = jnp.concatenate(cols, axis=1)


def _tc3(y2p, S2p, deg4, W2, b2):
    return pl.pallas_call(
        _tc3_body,
        out_shape=jax.ShapeDtypeStruct((NQ, 4 * D_OUT), jnp.float32),
    )(y2p, S2p, deg4, W2, b2.reshape(1, D_OUT))


def kernel(x, edge_index, W1, b1, gamma, beta, W2, b2):
    er = edge_index.reshape(2, NW, CPT, CHUNK)
    x4 = x.reshape(NQ, 4 * D_IN)
    zeros1 = jnp.zeros((NPAD,), jnp.float32)
    zeros2 = jnp.zeros((NPAD, D_HID), jnp.float32)

    degp = _deg_kernel(er, zeros1)
    deg4 = degp.reshape(NC, NP4, 4)
    y1p = _tc1(x4, W1, deg4)
    y1 = y1p.reshape(N, D_HID)
    S1 = _scatter_kernel(y1, er, zeros2)
    S1p = S1.reshape(NC, NP4, 128)
    y2p = _tc2(y1p, S1p, deg4, b1, gamma, beta)
    y2 = y2p.reshape(N, D_HID)
    S2 = _scatter_kernel(y2, er, zeros2)
    S2p = S2.reshape(NC, NP4, 128)
    out4 = _tc3(y2p, S2p, deg4, W2, b2)
    return out4.reshape(N, D_OUT)


# speedup vs baseline: 82.3110x; 1.0587x over previous
import functools

import jax
import jax.numpy as jnp
from jax import lax
from jax.experimental import pallas as pl
from jax.experimental.pallas import tpu as pltpu
from jax.experimental.pallas import tpu_sc as plsc

N = 10000
E = 320000
D_IN = 128
D_HID = 32
D_OUT = 128

NPAD = 10240
ROWS_PER_TILE = NPAD // 16
NP4 = NPAD // 4
NV4 = N // 4

NC = 2
NS = 16
NW = NC * NS
CH = 128
NCH = E // CH
BCPT = NCH // NW
XTRA = NCH - BCPT * NW
GK = 6
NG = BCPT // GK

_sc_mesh = plsc.VectorSubcoreMesh(core_axis_name="c", subcore_axis_name="s",
                                  num_cores=NC, num_subcores=NS)


def _slab_load(e3_hbm, idxb, w):
    lo = w * BCPT + jnp.minimum(w, XTRA)
    pltpu.sync_copy(e3_hbm.at[pl.ds(lo, BCPT)], idxb.at[pl.ds(0, BCPT)])

    @pl.when(w < XTRA)
    def _():
        pltpu.sync_copy(e3_hbm.at[pl.ds(lo + BCPT, 1)],
                        idxb.at[pl.ds(BCPT, 1)])


def _deg_body(e3_hbm, zeros1_hbm, degp_hbm, idxb, ones_v, sem, acc):
    c = lax.axis_index("c")
    s = lax.axis_index("s")
    w = c * NS + s
    for k in range(8):
        ones_v[pl.ds(k * 16, 16)] = jnp.ones((16,), jnp.float32)
    pltpu.sync_copy(zeros1_hbm.at[pl.ds(s * ROWS_PER_TILE, ROWS_PER_TILE)],
                    acc.at[pl.ds(s * ROWS_PER_TILE, ROWS_PER_TILE)])
    _slab_load(e3_hbm, idxb, w)
    plsc.subcore_barrier()

    def step(g, carry):
        ds = [pltpu.async_copy(ones_v, acc.at[idxb.at[g * GK + b, 1]], sem,
                               add=True) for b in range(GK)]
        for d in ds:
            d.wait()
        return carry

    lax.fori_loop(0, NG, step, 0)

    @pl.when(w < XTRA)
    def _():
        pltpu.async_copy(ones_v, acc.at[idxb.at[BCPT, 1]], sem,
                         add=True).wait()

    plsc.subcore_barrier()
    pltpu.sync_copy(acc.at[pl.ds(s * ROWS_PER_TILE, ROWS_PER_TILE)],
                    degp_hbm.at[c, pl.ds(s * ROWS_PER_TILE, ROWS_PER_TILE)])


_deg_kernel = functools.partial(
    pl.kernel,
    out_type=jax.ShapeDtypeStruct((NC, NPAD), jnp.float32),
    mesh=_sc_mesh,
    compiler_params=pltpu.CompilerParams(use_tc_tiling_on_sc=False),
    scratch_types=[
        pltpu.VMEM((BCPT + 1, 2, CH), jnp.int32),
        pltpu.VMEM((CH,), jnp.float32),
        pltpu.SemaphoreType.DMA,
        pltpu.VMEM_SHARED((NPAD,), jnp.float32),
    ],
)(_deg_body)


def _scatter_body(vals_hbm, e3_hbm, zeros2_hbm, out_hbm,
                  idxb, rows_v, gsem, ssem, acc):
    c = lax.axis_index("c")
    s = lax.axis_index("s")
    w = c * NS + s
    pltpu.sync_copy(zeros2_hbm.at[pl.ds(s * ROWS_PER_TILE, ROWS_PER_TILE)],
                    acc.at[pl.ds(s * ROWS_PER_TILE, ROWS_PER_TILE)])
    _slab_load(e3_hbm, idxb, w)
    plsc.subcore_barrier()

    def wait_g(bank, b):
        pltpu.make_async_copy(vals_hbm.at[idxb.at[0, 0]],
                              rows_v.at[bank, b], gsem).wait()

    def wait_s(bank, b):
        pltpu.make_async_copy(rows_v.at[bank, b],
                              acc.at[idxb.at[0, 1]], ssem).wait()

    for b in range(GK):
        pltpu.async_copy(vals_hbm.at[idxb.at[b, 0]], rows_v.at[0, b], gsem)

    def step(g, carry):
        p = lax.rem(g, 2)
        po = 1 - p

        @pl.when(g > 0)
        def _():
            for b in range(GK):
                wait_s(po, b)

        @pl.when(g < NG - 1)
        def _():
            for b in range(GK):
                pltpu.async_copy(vals_hbm.at[idxb.at[(g + 1) * GK + b, 0]],
                                 rows_v.at[po, b], gsem)

        for b in range(GK):
            wait_g(p, b)
            pltpu.async_copy(rows_v.at[p, b],
                             acc.at[idxb.at[g * GK + b, 1]],
                             ssem, add=True)
        return carry

    lax.fori_loop(0, NG, step, 0)
    for b in range(GK):
        wait_s((NG - 1) % 2, b)

    @pl.when(w < XTRA)
    def _():
        pltpu.async_copy(vals_hbm.at[idxb.at[BCPT, 0]],
                         rows_v.at[0, 0], gsem).wait()
        pltpu.async_copy(rows_v.at[0, 0], acc.at[idxb.at[BCPT, 1]],
                         ssem, add=True).wait()

    plsc.subcore_barrier()
    pltpu.sync_copy(acc.at[pl.ds(s * ROWS_PER_TILE, ROWS_PER_TILE)],
                    out_hbm.at[c, pl.ds(s * ROWS_PER_TILE, ROWS_PER_TILE)])


_scatter_kernel = functools.partial(
    pl.kernel,
    out_type=jax.ShapeDtypeStruct((NC, NPAD, D_HID), jnp.float32),
    mesh=_sc_mesh,
    compiler_params=pltpu.CompilerParams(use_tc_tiling_on_sc=False),
    scratch_types=[
        pltpu.VMEM((BCPT + 1, 2, CH), jnp.int32),
        pltpu.VMEM((2, GK, CH, D_HID), jnp.float32),
        pltpu.SemaphoreType.DMA,
        pltpu.SemaphoreType.DMA,
        pltpu.VMEM_SHARED((NPAD, D_HID), jnp.float32),
    ],
)(_scatter_body)


NQ = N // 4


def _dinv4(deg4):
    return lax.rsqrt(deg4 + 1.0)


def _tc1_body(x4_ref, w1_ref, deg4_ref, y1p_ref):
    dinv = _dinv4(deg4_ref[0][:NQ] + deg4_ref[1][:NQ])
    cols = []
    for a in range(4):
        xw = jnp.dot(x4_ref[:, 128 * a:128 * (a + 1)], w1_ref[...],
                     preferred_element_type=jnp.float32)
        cols.append(xw * jnp.broadcast_to(dinv[:, a:a + 1], (NQ, D_HID)))
    y1p_ref[...] = jnp.concatenate(cols, axis=1)


def _tc1(x4, W1, deg4):
    return pl.pallas_call(
        _tc1_body,
        out_shape=jax.ShapeDtypeStruct((NQ, 128), jnp.float32),
    )(x4, W1, deg4)


def _tc2_body(y1p_ref, s1p_ref, deg4_ref, b1_ref, g_ref, bt_ref, y2p_ref):
    dinv = _dinv4(deg4_ref[0][:NQ] + deg4_ref[1][:NQ])
    dinvp = jnp.concatenate(
        [jnp.broadcast_to(dinv[:, a:a + 1], (NQ, D_HID)) for a in range(4)],
        axis=1)
    sp = s1p_ref[0][:NQ] + s1p_ref[1][:NQ] + y1p_ref[...]
    h_pre = dinvp * sp + b1_ref[...]
    s4 = jnp.sum(h_pre, axis=0, keepdims=True)
    mean = (s4[:, 0:32] + s4[:, 32:64] + s4[:, 64:96] + s4[:, 96:128]) / N
    meanp = jnp.concatenate([mean] * 4, axis=1)
    d = h_pre - meanp
    v4 = jnp.sum(d * d, axis=0, keepdims=True)
    var = (v4[:, 0:32] + v4[:, 32:64] + v4[:, 64:96] + v4[:, 96:128]) / N
    varp = jnp.concatenate([var] * 4, axis=1)
    hn = d * lax.rsqrt(varp + 1e-5) * g_ref[...] + bt_ref[...]
    y2p_ref[...] = jnp.maximum(hn, 0.0) * dinvp


def _tc2(y1p, S1p, deg4, b1, gamma, beta):
    t = lambda v: jnp.tile(v, 4).reshape(1, 128)
    return pl.pallas_call(
        _tc2_body,
        out_shape=jax.ShapeDtypeStruct((NQ, 128), jnp.float32),
    )(y1p, S1p, deg4, t(b1), t(gamma), t(beta))


def _tc3_body(y2p_ref, s2p_ref, deg4_ref, w2_ref, b2_ref, out4_ref):
    dinv = _dinv4(deg4_ref[0][:NQ] + deg4_ref[1][:NQ])
    cols = []
    for a in range(4):
        sl = slice(32 * a, 32 * (a + 1))
        agg = y2p_ref[:, sl] + s2p_ref[0][:NQ, sl] + s2p_ref[1][:NQ, sl]
        agg = agg * jnp.broadcast_to(dinv[:, a:a + 1], (NQ, D_HID))
        cols.append(jnp.dot(agg, w2_ref[...],
                            preferred_element_type=jnp.float32)
                    + b2_ref[...])
    out4_ref[...] = jnp.concatenate(cols, axis=1)


def _tc3(y2p, S2p, deg4, W2, b2):
    return pl.pallas_call(
        _tc3_body,
        out_shape=jax.ShapeDtypeStruct((NQ, 4 * D_OUT), jnp.float32),
    )(y2p, S2p, deg4, W2, b2.reshape(1, D_OUT))


def kernel(x, edge_index, W1, b1, gamma, beta, W2, b2):
    e3 = edge_index.reshape(2, NCH, CH).transpose(1, 0, 2)
    x4 = x.reshape(NQ, 4 * D_IN)
    zeros1 = jnp.zeros((NPAD,), jnp.float32)
    zeros2 = jnp.zeros((NPAD, D_HID), jnp.float32)

    degp = _deg_kernel(e3, zeros1)
    deg4 = degp.reshape(NC, NP4, 4)
    y1p = _tc1(x4, W1, deg4)
    y1 = y1p.reshape(N, D_HID)
    S1 = _scatter_kernel(y1, e3, zeros2)
    S1p = S1.reshape(NC, NP4, 128)
    y2p = _tc2(y1p, S1p, deg4, b1, gamma, beta)
    y2 = y2p.reshape(N, D_HID)
    S2 = _scatter_kernel(y2, e3, zeros2)
    S2p = S2.reshape(NC, NP4, 128)
    out4 = _tc3(y2p, S2p, deg4, W2, b2)
    return out4.reshape(N, D_OUT)
